# Initial kernel scaffold; baseline (speedup 1.0000x reference)
import jax
import jax.numpy as jnp
from jax.experimental import pallas as pl


def kernel(positions, sizes, macro_mask):
    raise NotImplementedError("write your pallas kernel here")


# speedup vs baseline: 62.4666x; 62.4666x over previous
import functools

import jax
import jax.numpy as jnp
import numpy as np
from jax.experimental import pallas as pl
from jax.experimental.pallas import tpu as pltpu

_G = 256
_SIGMA = 2.0
_TARGET = 1.0


def _build_smooth_matrix():
    k_size = int(6 * _SIGMA) | 1
    x = np.arange(k_size, dtype=np.float32) - k_size // 2
    k1 = np.exp(-(x ** 2) / (2.0 * _SIGMA ** 2))
    w = (k1 / k1.sum()).astype(np.float64)
    pad = k_size // 2
    s = np.zeros((_G, _G), dtype=np.float64)
    for t in range(k_size):
        off = t - pad
        for g in range(_G):
            i = g + off
            if i < 0:
                i = -i
            elif i >= _G:
                i = 2 * _G - 2 - i
            s[g, i] += w[t]
    return s.astype(np.float32)


_SMOOTH = _build_smooth_matrix()


def _body(px_ref, py_ref, ax_ref, ay_ref, s_ref, den_ref, loss_ref):
    g = _G
    gx = (px_ref[0] + 1.0) * ((g - 1) / 2.0)
    gy = (py_ref[0] + 1.0) * ((g - 1) / 2.0)
    v = gx.shape[1]
    coords = jax.lax.broadcasted_iota(jnp.int32, (g, v), 0).astype(jnp.float32)
    x_in = jax.nn.sigmoid(ax_ref[0] - 2.0 * jnp.abs(coords - gx))
    y_in = jax.nn.sigmoid(ay_ref[0] - 2.0 * jnp.abs(coords - gy))
    d = jax.lax.dot_general(y_in, x_in, (((1,), (1,)), ((), ())),
                            preferred_element_type=jnp.float32)
    s = s_ref[...]
    t = jnp.dot(s, d, preferred_element_type=jnp.float32)
    out = jax.lax.dot_general(t, s, (((1,), (1,)), ((), ())),
                              preferred_element_type=jnp.float32)
    den_ref[...] = out[None]
    ov = jnp.maximum(out - _TARGET, 0.0)
    part = jnp.sum(ov * ov, axis=0, keepdims=True)
    loss_ref[...] = (part[:, :128] + part[:, 128:]).reshape(1, 1, 128)


@jax.jit
def kernel(positions, sizes, macro_mask):
    b, v, _ = positions.shape
    g = _G
    px = positions[:, :, 0].reshape(b, 1, v)
    py = positions[:, :, 1].reshape(b, 1, v)
    mask = macro_mask
    ax = (sizes[:, 0] * (g / 2.0)).reshape(1, 1, v)
    ay = jnp.where(mask, sizes[:, 1] * (g / 2.0), -1e9).reshape(1, 1, v)
    smooth = jnp.asarray(_SMOOTH)

    den, loss_part = pl.pallas_call(
        _body,
        grid=(b,),
        in_specs=[
            pl.BlockSpec((1, 1, v), lambda i: (i, 0, 0)),
            pl.BlockSpec((1, 1, v), lambda i: (i, 0, 0)),
            pl.BlockSpec((1, 1, v), lambda i: (0, 0, 0)),
            pl.BlockSpec((1, 1, v), lambda i: (0, 0, 0)),
            pl.BlockSpec((g, g), lambda i: (0, 0)),
        ],
        out_specs=[
            pl.BlockSpec((1, g, g), lambda i: (i, 0, 0)),
            pl.BlockSpec((1, 1, 128), lambda i: (i, 0, 0)),
        ],
        out_shape=[
            jax.ShapeDtypeStruct((b, g, g), jnp.float32),
            jax.ShapeDtypeStruct((b, 1, 128), jnp.float32),
        ],
        compiler_params=pltpu.CompilerParams(
            dimension_semantics=("parallel",),
        ),
    )(px, py, ax, ay, smooth)

    density = den.reshape(b, 1, g, g)
    overflow_loss = jnp.sum(loss_part) / (b * g * g)
    return density, overflow_loss
